# baseline (speedup 1.0000x reference)
import functools

import jax
import jax.numpy as jnp
from jax import lax
from jax.experimental import pallas as pl
from jax.experimental.pallas import tpu as pltpu
from jax.experimental.pallas import tpu_sc as plsc

_STATIC_VARIABLES = 26
_EMBEDDING_DIM = 64
_BATCH = 4096
_OUT_D = _STATIC_VARIABLES * _EMBEDDING_DIM
_TOTAL = _BATCH * _STATIC_VARIABLES
_NC = 2
_NS = 16
_NW = _NC * _NS
_CHUNK = 416
_NCHUNK = _TOTAL // (_NW * _CHUNK)
_ROWS_W = _BATCH // _NW
_ROWS_CH = _ROWS_W // _NCHUNK
_VPR = _OUT_D // 16

_mesh = plsc.VectorSubcoreMesh(core_axis_name="c", subcore_axis_name="s")


@functools.partial(
    pl.kernel,
    mesh=_mesh,
    out_type=jax.ShapeDtypeStruct((_BATCH, _OUT_D), jnp.float32),
    scratch_types=[
        pltpu.VMEM((_NCHUNK, _CHUNK), jnp.int32),
        pltpu.VMEM((_CHUNK, _EMBEDDING_DIM), jnp.float32),
        pltpu.VMEM((_CHUNK, _EMBEDDING_DIM), jnp.float32),
        pltpu.VMEM((_ROWS_CH, _OUT_D), jnp.float32),
        pltpu.VMEM((_ROWS_CH, _OUT_D), jnp.float32),
        pltpu.SemaphoreType.DMA,
        pltpu.SemaphoreType.DMA,
        pltpu.SemaphoreType.DMA,
        pltpu.SemaphoreType.DMA,
        pltpu.VMEM_SHARED((_STATIC_VARIABLES, _EMBEDDING_DIM), jnp.float32),
    ],
    compiler_params=pltpu.CompilerParams(use_tc_tiling_on_sc=False),
)
def _emb_lookup(idx_hbm, table_hbm, out_hbm, idx_v, bg0, bg1, bw0, bw1,
                g0, g1, w0, w1, tab_sh):
    sid = lax.axis_index("s")
    wid = sid * _NC + lax.axis_index("c")
    row0 = wid * _ROWS_W
    @pl.when(sid == 0)
    def _():
        pltpu.sync_copy(table_hbm, tab_sh)

    pltpu.sync_copy(idx_hbm.at[wid], idx_v)
    plsc.subcore_barrier()

    bufg = (bg0, bg1)
    bufw = (bw0, bw1)
    gsems = (g0, g1)
    wsems = (w0, w1)

    def gather(j):
        b = j % 2
        return pltpu.async_copy(tab_sh.at[idx_v.at[j]], bufg[b], gsems[b])

    def repack(b):
        src, dst = bufg[b], bufw[b]

        def rowcopy(r, carry):
            rbase = r * (_VPR // 4)
            for u in range(_VPR):
                dst[r, pl.ds(u * 16, 16)] = src[rbase + u // 4, pl.ds((u % 4) * 16, 16)]
            return carry

        lax.fori_loop(0, _ROWS_CH, rowcopy, 0)

    gathers = [None] * _NCHUNK
    writes = [None] * _NCHUNK
    gathers[0] = gather(0)
    for j in range(_NCHUNK):
        b = j % 2
        if j + 1 < _NCHUNK:
            gathers[j + 1] = gather(j + 1)
        gathers[j].wait()
        if j >= 2:
            writes[j - 2].wait()
        repack(b)
        writes[j] = pltpu.async_copy(
            bufw[b],
            out_hbm.at[pl.ds(row0 + j * _ROWS_CH, _ROWS_CH)],
            wsems[b],
        )
    writes[_NCHUNK - 2].wait()
    writes[_NCHUNK - 1].wait()


def kernel(static_input, table):
    idx = static_input.astype(jnp.int32).reshape(_NW, _NCHUNK, _CHUNK)
    return _emb_lookup(idx, table.astype(jnp.float32))


# speedup vs baseline: 1.7841x; 1.7841x over previous
import functools

import jax
import jax.numpy as jnp
from jax import lax
from jax.experimental import pallas as pl
from jax.experimental.pallas import tpu as pltpu
from jax.experimental.pallas import tpu_sc as plsc

_V = 26
_E = 64
_BATCH = 4096
_OUT_D = _V * _E
_NP = _V // 2
_NC = 2
_NS = 16
_NW = _NC * _NS
_ROWS_W = _BATCH // _NW
_TR_W = _ROWS_W // 8
_TR_CH = 2
_NCHUNK = _TR_W // _TR_CH
_ROWS_CH = 8 * _TR_CH

_mesh = plsc.VectorSubcoreMesh(core_axis_name="c", subcore_axis_name="s")


@functools.partial(
    pl.kernel,
    mesh=_mesh,
    out_type=jax.ShapeDtypeStruct((_BATCH, _OUT_D), jnp.float32),
    scratch_types=[
        pltpu.VMEM((_TR_W, 128), jnp.int32),
        pltpu.VMEM((_ROWS_CH, _OUT_D), jnp.float32),
        pltpu.VMEM((_ROWS_CH, _OUT_D), jnp.float32),
        pltpu.SemaphoreType.DMA,
        pltpu.SemaphoreType.DMA,
        pltpu.SemaphoreType.DMA,
        pltpu.SemaphoreType.DMA,
        pltpu.VMEM_SHARED((_V * _V, 128), jnp.float32),
    ],
)
def _emb_lookup(idx_hbm, tab2_hbm, out_hbm, idx_v, bw0, bw1, g0, g1, w0, w1, tab_sh):
    sid = lax.axis_index("s")
    wid = sid * _NC + lax.axis_index("c")
    row0 = wid * _ROWS_W
    @pl.when(sid == 0)
    def _():
        pltpu.sync_copy(tab2_hbm, tab_sh)

    pltpu.sync_copy(idx_hbm.at[wid], idx_v)
    plsc.subcore_barrier()

    bufw = (bw0, bw1)
    gsems = (g0, g1)
    wsems = (w0, w1)

    def gather_chunk(j):
        b = j % 2
        cps = []
        for tt in range(_TR_CH):
            for c in range(_NP):
                cps.append(pltpu.async_copy(
                    tab_sh.at[idx_v.at[j * _TR_CH + tt, pl.ds(c * 8, 8)]],
                    bufw[b].at[pl.ds(tt * 8, 8), pl.ds(c * 128, 128)],
                    gsems[b],
                ))
        return cps

    gathers = [None] * _NCHUNK
    writes = [None] * _NCHUNK
    gathers[0] = gather_chunk(0)
    for j in range(_NCHUNK):
        b = j % 2
        for cp in gathers[j]:
            cp.wait()
        writes[j] = pltpu.async_copy(
            bufw[b],
            out_hbm.at[pl.ds(row0 + j * _ROWS_CH, _ROWS_CH)],
            wsems[b],
        )
        if j + 1 < _NCHUNK:
            if j >= 1:
                writes[j - 1].wait()
            gathers[j + 1] = gather_chunk(j + 1)
    writes[_NCHUNK - 2].wait()
    writes[_NCHUNK - 1].wait()


def kernel(static_input, table):
    tab2 = jnp.concatenate(
        [
            jnp.broadcast_to(table[:, None, :], (_V, _V, _E)),
            jnp.broadcast_to(table[None, :, :], (_V, _V, _E)),
        ],
        axis=-1,
    ).reshape(_V * _V, 2 * _E)
    si = static_input.astype(jnp.int32)
    pairs = si[:, 0::2] * _V + si[:, 1::2]
    pairs = pairs.reshape(_NW, _TR_W, 8, _NP).transpose(0, 1, 3, 2)
    pairs = pairs.reshape(_NW, _TR_W, _NP * 8)
    pairs = jnp.pad(pairs, ((0, 0), (0, 0), (0, 128 - _NP * 8)))
    return _emb_lookup(pairs, tab2)
